# baseline (speedup 1.0000x reference)
import functools

import jax
import jax.numpy as jnp
from jax import lax
from jax.experimental import pallas as pl
from jax.experimental.pallas import tpu as pltpu
from jax.experimental.pallas import tpu_sc as plsc

B = 16
PS = 4096
C = 16
N = 65536

IN_WORDS = B * PS * 2 * C
OUT_WORDS = B * N * C
CHUNK_IN = 1024
CHUNK_OUT = 16384
NCHUNK = 32
STAGE_WORDS = 2 * 16 * CHUNK_OUT


@functools.partial(
    pl.kernel,
    mesh=plsc.VectorSubcoreMesh(core_axis_name="c", subcore_axis_name="s"),
    out_type=jax.ShapeDtypeStruct((OUT_WORDS,), jnp.float32),
    scratch_types=[
        pltpu.VMEM((CHUNK_IN,), jnp.float32),
        pltpu.VMEM((CHUNK_IN,), jnp.float32),
        pltpu.VMEM((CHUNK_IN,), jnp.float32),
        pltpu.VMEM((CHUNK_IN,), jnp.float32),
        pltpu.VMEM((CHUNK_IN,), jnp.int32),
        pltpu.VMEM((CHUNK_IN,), jnp.int32),
        pltpu.VMEM((CHUNK_OUT,), jnp.float32),
        pltpu.VMEM((CHUNK_OUT,), jnp.float32),
        pltpu.VMEM_SHARED((STAGE_WORDS,), jnp.float32),
        pltpu.SemaphoreType.DMA,
        pltpu.SemaphoreType.DMA,
        pltpu.SemaphoreType.DMA,
        pltpu.SemaphoreType.DMA,
        pltpu.SemaphoreType.DMA,
        pltpu.SemaphoreType.DMA,
        pltpu.SemaphoreType.DMA,
    ],
    compiler_params=pltpu.CompilerParams(needs_layout_passes=False),
)
def _velvet_sc(in_hbm, out_hbm, sgn_a, frac_a, sgn_b, frac_b, offs_a, offs_b,
               out_a, out_b, stage, sem_a, semx0, semx1, semy0, semy1,
               isem_a, isem_b):
    b = lax.axis_index("s")
    cr = lax.axis_index("c")
    w = b * 2 + cr
    sgn_w0 = (b * 4 + cr) * 256 * 128
    frac_w0 = sgn_w0 + 512 * 128
    out_w0 = w * 4096 * 128

    semx = (semx0, semx1)
    semy = (semy0, semy1)

    iota = lax.iota(jnp.int32, 16)
    lane_off = lax.shift_left(iota & 8, 7) + lax.shift_left(iota & 7, 4)
    zvec = jnp.zeros((16,), jnp.float32)

    def zero_buf(buf):
        def zrow(r, _):
            base = r * 128
            for cc in range(8):
                buf[pl.ds(base + cc * 16, 16)] = zvec
            return 0
        lax.fori_loop(0, 128, zrow, 0)

    def in_issue(k, sgn_v, frac_v, sem):
        pltpu.async_copy(in_hbm.at[pl.ds(sgn_w0 + CHUNK_IN * k, CHUNK_IN)], sgn_v, sem)
        pltpu.async_copy(in_hbm.at[pl.ds(frac_w0 + CHUNK_IN * k, CHUNK_IN)], frac_v, sem)

    def in_wait(k, sgn_v, frac_v, sem):
        pltpu.make_async_copy(in_hbm.at[pl.ds(sgn_w0 + CHUNK_IN * k, CHUNK_IN)], sgn_v, sem).wait()
        pltpu.make_async_copy(in_hbm.at[pl.ds(frac_w0 + CHUNK_IN * k, CHUNK_IN)], frac_v, sem).wait()

    def value_pass(sgn_v, frac_v, out_v, offs_v):
        def mbody(m, _):
            sbase = m * 128
            for gg in range(8):
                sgn = sgn_v[pl.ds(sbase + gg * 16, 16)]
                frac = frac_v[pl.ds(sbase + gg * 16, 16)]
                idx = (frac * 15.0).astype(jnp.int32)
                offs = (lane_off + (sbase + gg * 2048)) + idx
                offs_v[pl.ds(sbase + gg * 16, 16)] = offs
                plsc.store_scatter(out_v, [offs], sgn)
            return 0
        lax.fori_loop(0, 8, mbody, 0)

    def rescatter_zeros(offs_v, out_v):
        def rbody(m, _):
            base = m * 128
            for gg in range(8):
                offs = offs_v[pl.ds(base + gg * 16, 16)]
                plsc.store_scatter(out_v, [offs], zvec)
            return 0
        lax.fori_loop(0, 8, rbody, 0)

    def out_slice(k):
        return out_hbm.at[pl.ds(out_w0 + CHUNK_OUT * k, CHUNK_OUT)]

    def stage_slice(s):
        return stage.at[pl.ds((s * 16 + b) * CHUNK_OUT, CHUNK_OUT)]

    def a_chunk(k, first=False, prefetch=True):
        if not first:
            pltpu.make_async_copy(out_a, out_slice(k - 2), sem_a).wait()
            rescatter_zeros(offs_a, out_a)
        in_wait(k, sgn_a, frac_a, isem_a)
        value_pass(sgn_a, frac_a, out_a, offs_a)
        if prefetch:
            in_issue(k + 2, sgn_a, frac_a, isem_a)
        pltpu.async_copy(out_a, out_slice(k), sem_a)

    def b_chunk(k, s, first=False, second=False, prefetch=True):
        os = 1 - s
        if not first:
            pltpu.make_async_copy(out_b, stage_slice(os), semx[os]).wait()
            pltpu.async_copy(stage_slice(os), out_slice(k - 2), semy[os])
            rescatter_zeros(offs_b, out_b)
        in_wait(k, sgn_b, frac_b, isem_b)
        value_pass(sgn_b, frac_b, out_b, offs_b)
        if prefetch:
            in_issue(k + 2, sgn_b, frac_b, isem_b)
        if not (first or second):
            pltpu.make_async_copy(stage_slice(s), out_slice(k - 4), semy[s]).wait()
        pltpu.async_copy(out_b, stage_slice(s), semx[s])

    in_issue(0, sgn_a, frac_a, isem_a)
    in_issue(1, sgn_b, frac_b, isem_b)
    zero_buf(out_a)
    a_chunk(0, first=True)
    zero_buf(out_b)
    b_chunk(1, 0, first=True)
    a_chunk(2)
    b_chunk(3, 1, second=True)

    def quad(u, _):
        k = 4 * u
        a_chunk(k)
        b_chunk(k + 1, 0)
        a_chunk(k + 2)
        b_chunk(k + 3, 1)
        return 0

    lax.fori_loop(1, NCHUNK // 4 - 1, quad, 0)

    a_chunk(NCHUNK - 4)
    b_chunk(NCHUNK - 3, 0)
    a_chunk(NCHUNK - 2, prefetch=False)
    b_chunk(NCHUNK - 1, 1, prefetch=False)

    pltpu.make_async_copy(out_b, stage_slice(1), semx1).wait()
    pltpu.async_copy(stage_slice(1), out_slice(NCHUNK - 1), semy1)
    pltpu.make_async_copy(out_a, out_slice(NCHUNK - 2), sem_a).wait()
    pltpu.make_async_copy(stage_slice(0), out_slice(NCHUNK - 3), semy0).wait()
    pltpu.make_async_copy(stage_slice(1), out_slice(NCHUNK - 1), semy1).wait()


def kernel(inputs):
    flat_in = (inputs.transpose(0, 2, 1)
               .reshape(B, 4, 8, 32, 128)
               .transpose(0, 1, 3, 2, 4)
               .reshape(IN_WORDS))
    flat_out = _velvet_sc(flat_in)
    return (flat_out.reshape(B, 2, 512, 8, 128)
            .transpose(0, 2, 4, 1, 3)
            .reshape(B, N, C))


# speedup vs baseline: 1.0439x; 1.0439x over previous
import functools

import jax
import jax.numpy as jnp
from jax import lax
from jax.experimental import pallas as pl
from jax.experimental.pallas import tpu as pltpu
from jax.experimental.pallas import tpu_sc as plsc

B = 16
PS = 4096
C = 16
N = 65536

IN_WORDS = B * PS * 2 * C
OUT_WORDS = B * N * C
CHUNK_IN = 1024
CHUNK_OUT = 16384
NCHUNK = 32
STAGE_WORDS = 2 * 16 * CHUNK_OUT


@functools.partial(
    pl.kernel,
    mesh=plsc.VectorSubcoreMesh(core_axis_name="c", subcore_axis_name="s"),
    out_type=jax.ShapeDtypeStruct((OUT_WORDS,), jnp.float32),
    scratch_types=[
        pltpu.VMEM((CHUNK_IN,), jnp.float32),
        pltpu.VMEM((CHUNK_IN,), jnp.float32),
        pltpu.VMEM((CHUNK_IN,), jnp.float32),
        pltpu.VMEM((CHUNK_IN,), jnp.float32),
        pltpu.VMEM((CHUNK_IN,), jnp.int32),
        pltpu.VMEM((CHUNK_IN,), jnp.int32),
        pltpu.VMEM((CHUNK_OUT,), jnp.float32),
        pltpu.VMEM((CHUNK_OUT,), jnp.float32),
        pltpu.VMEM_SHARED((STAGE_WORDS,), jnp.float32),
        pltpu.SemaphoreType.DMA,
        pltpu.SemaphoreType.DMA,
        pltpu.SemaphoreType.DMA,
        pltpu.SemaphoreType.DMA,
        pltpu.SemaphoreType.DMA,
        pltpu.SemaphoreType.DMA,
        pltpu.SemaphoreType.DMA,
    ],
    compiler_params=pltpu.CompilerParams(needs_layout_passes=False),
)
def _velvet_sc(in_hbm, out_hbm, sgn_a, frac_a, sgn_b, frac_b, offs_a, offs_b,
               out_a, out_b, stage, sem_a, semx0, semx1, semy0, semy1,
               isem_a, isem_b):
    b = lax.axis_index("s")
    cr = lax.axis_index("c")
    w = b * 2 + cr
    sgn_w0 = (b * 4 + cr) * 256 * 128
    frac_w0 = sgn_w0 + 512 * 128
    out_w0 = w * 4096 * 128

    semx = (semx0, semx1)
    semy = (semy0, semy1)

    iota = lax.iota(jnp.int32, 16)
    lane_off = lax.shift_left(iota & 8, 7) + lax.shift_left(iota & 7, 4)
    zvec = jnp.zeros((16,), jnp.float32)

    def zero_buf(buf):
        def zrow(r, _):
            base = r * 128
            for cc in range(8):
                buf[pl.ds(base + cc * 16, 16)] = zvec
            return 0
        lax.fori_loop(0, 128, zrow, 0)

    def in_issue(k, sgn_v, frac_v, sem):
        pltpu.async_copy(in_hbm.at[pl.ds(sgn_w0 + CHUNK_IN * k, CHUNK_IN)], sgn_v, sem)
        pltpu.async_copy(in_hbm.at[pl.ds(frac_w0 + CHUNK_IN * k, CHUNK_IN)], frac_v, sem)

    def in_wait(k, sgn_v, frac_v, sem):
        pltpu.make_async_copy(in_hbm.at[pl.ds(sgn_w0 + CHUNK_IN * k, CHUNK_IN)], sgn_v, sem).wait()
        pltpu.make_async_copy(in_hbm.at[pl.ds(frac_w0 + CHUNK_IN * k, CHUNK_IN)], frac_v, sem).wait()

    def value_pass(sgn_v, frac_v, out_v, offs_v):
        @plsc.parallel_loop(0, 8, unroll=2)
        def mbody(m):
            sbase = m * 128
            for gg in range(8):
                sgn = sgn_v[pl.ds(sbase + gg * 16, 16)]
                frac = frac_v[pl.ds(sbase + gg * 16, 16)]
                idx = (frac * 15.0).astype(jnp.int32)
                offs = (lane_off + (sbase + gg * 2048)) + idx
                offs_v[pl.ds(sbase + gg * 16, 16)] = offs
                plsc.store_scatter(out_v, [offs], sgn)

    def rescatter_zeros(offs_v, out_v):
        @plsc.parallel_loop(0, 8, unroll=2)
        def rbody(m):
            base = m * 128
            for gg in range(8):
                offs = offs_v[pl.ds(base + gg * 16, 16)]
                plsc.store_scatter(out_v, [offs], zvec)

    def out_slice(k):
        return out_hbm.at[pl.ds(out_w0 + CHUNK_OUT * k, CHUNK_OUT)]

    def stage_slice(s):
        return stage.at[pl.ds((s * 16 + b) * CHUNK_OUT, CHUNK_OUT)]

    def a_chunk(k, first=False, prefetch=True):
        if not first:
            pltpu.make_async_copy(out_a, out_slice(k - 2), sem_a).wait()
            rescatter_zeros(offs_a, out_a)
        in_wait(k, sgn_a, frac_a, isem_a)
        value_pass(sgn_a, frac_a, out_a, offs_a)
        if prefetch:
            in_issue(k + 2, sgn_a, frac_a, isem_a)
        pltpu.async_copy(out_a, out_slice(k), sem_a)

    def b_chunk(k, s, first=False, second=False, prefetch=True):
        os = 1 - s
        if not first:
            pltpu.make_async_copy(out_b, stage_slice(os), semx[os]).wait()
            pltpu.async_copy(stage_slice(os), out_slice(k - 2), semy[os])
            rescatter_zeros(offs_b, out_b)
        in_wait(k, sgn_b, frac_b, isem_b)
        value_pass(sgn_b, frac_b, out_b, offs_b)
        if prefetch:
            in_issue(k + 2, sgn_b, frac_b, isem_b)
        if not (first or second):
            pltpu.make_async_copy(stage_slice(s), out_slice(k - 4), semy[s]).wait()
        pltpu.async_copy(out_b, stage_slice(s), semx[s])

    in_issue(0, sgn_a, frac_a, isem_a)
    in_issue(1, sgn_b, frac_b, isem_b)
    zero_buf(out_a)
    a_chunk(0, first=True)
    zero_buf(out_b)
    b_chunk(1, 0, first=True)
    a_chunk(2)
    b_chunk(3, 1, second=True)

    def quad(u, _):
        k = 4 * u
        a_chunk(k)
        b_chunk(k + 1, 0)
        a_chunk(k + 2)
        b_chunk(k + 3, 1)
        return 0

    lax.fori_loop(1, NCHUNK // 4 - 1, quad, 0)

    a_chunk(NCHUNK - 4)
    b_chunk(NCHUNK - 3, 0)
    a_chunk(NCHUNK - 2, prefetch=False)
    b_chunk(NCHUNK - 1, 1, prefetch=False)

    pltpu.make_async_copy(out_b, stage_slice(1), semx1).wait()
    pltpu.async_copy(stage_slice(1), out_slice(NCHUNK - 1), semy1)
    pltpu.make_async_copy(out_a, out_slice(NCHUNK - 2), sem_a).wait()
    pltpu.make_async_copy(stage_slice(0), out_slice(NCHUNK - 3), semy0).wait()
    pltpu.make_async_copy(stage_slice(1), out_slice(NCHUNK - 1), semy1).wait()


def kernel(inputs):
    flat_in = (inputs.transpose(0, 2, 1)
               .reshape(B, 4, 8, 32, 128)
               .transpose(0, 1, 3, 2, 4)
               .reshape(IN_WORDS))
    flat_out = _velvet_sc(flat_in)
    return (flat_out.reshape(B, 2, 512, 8, 128)
            .transpose(0, 2, 4, 1, 3)
            .reshape(B, N, C))


# speedup vs baseline: 1.1555x; 1.1069x over previous
import functools

import jax
import jax.numpy as jnp
from jax import lax
from jax.experimental import pallas as pl
from jax.experimental.pallas import tpu as pltpu
from jax.experimental.pallas import tpu_sc as plsc

B = 16
PS = 4096
C = 16
N = 65536

IN_WORDS = B * PS * 2 * C
OUT_WORDS = B * N * C
CHUNK_IN = 1024
CHUNK_OUT = 16384
NCHUNK = 32
STAGE_WORDS = 2 * 16 * CHUNK_OUT


@functools.partial(
    pl.kernel,
    mesh=plsc.VectorSubcoreMesh(core_axis_name="c", subcore_axis_name="s"),
    out_type=jax.ShapeDtypeStruct((OUT_WORDS,), jnp.float32),
    scratch_types=[
        pltpu.VMEM((CHUNK_IN,), jnp.float32),
        pltpu.VMEM((CHUNK_IN,), jnp.float32),
        pltpu.VMEM((CHUNK_IN,), jnp.float32),
        pltpu.VMEM((CHUNK_IN,), jnp.float32),
        pltpu.VMEM((CHUNK_IN,), jnp.int32),
        pltpu.VMEM((CHUNK_IN,), jnp.int32),
        pltpu.VMEM((CHUNK_OUT,), jnp.float32),
        pltpu.VMEM((CHUNK_OUT,), jnp.float32),
        pltpu.VMEM_SHARED((STAGE_WORDS,), jnp.float32),
        pltpu.SemaphoreType.DMA,
        pltpu.SemaphoreType.DMA,
        pltpu.SemaphoreType.DMA,
        pltpu.SemaphoreType.DMA,
        pltpu.SemaphoreType.DMA,
        pltpu.SemaphoreType.DMA,
        pltpu.SemaphoreType.DMA,
    ],
    compiler_params=pltpu.CompilerParams(needs_layout_passes=False),
)
def _velvet_sc(in_hbm, out_hbm, sgn_a, frac_a, sgn_b, frac_b, offs_a, offs_b,
               out_a, out_b, stage, sem_a, semx0, semx1, semy0, semy1,
               isem_a, isem_b):
    b = lax.axis_index("s")
    cr = lax.axis_index("c")
    w = b * 2 + cr
    sgn_w0 = (b * 4 + cr) * 256 * 128
    frac_w0 = sgn_w0 + 512 * 128
    out_w0 = w * 4096 * 128

    semx = (semx0, semx1)
    semy = (semy0, semy1)

    iota = lax.iota(jnp.int32, 16)
    lane_off = lax.shift_left(iota & 8, 7) + lax.shift_left(iota & 7, 4)
    zvec = jnp.zeros((16,), jnp.float32)

    def zero_buf(buf):
        def zrow(r, _):
            base = r * 128
            for cc in range(8):
                buf[pl.ds(base + cc * 16, 16)] = zvec
            return 0
        lax.fori_loop(0, 128, zrow, 0)

    def in_issue(k, sgn_v, frac_v, sem):
        pltpu.async_copy(in_hbm.at[pl.ds(sgn_w0 + CHUNK_IN * k, CHUNK_IN)], sgn_v, sem)
        pltpu.async_copy(in_hbm.at[pl.ds(frac_w0 + CHUNK_IN * k, CHUNK_IN)], frac_v, sem)

    def in_wait(k, sgn_v, frac_v, sem):
        pltpu.make_async_copy(in_hbm.at[pl.ds(sgn_w0 + CHUNK_IN * k, CHUNK_IN)], sgn_v, sem).wait()
        pltpu.make_async_copy(in_hbm.at[pl.ds(frac_w0 + CHUNK_IN * k, CHUNK_IN)], frac_v, sem).wait()

    def init_offs(offs_v):
        def irow(m, _):
            sbase = m * 128
            for gg in range(8):
                offs_v[pl.ds(sbase + gg * 16, 16)] = lane_off + (sbase + gg * 2048)
            return 0
        lax.fori_loop(0, 8, irow, 0)

    def merged_pass(sgn_v, frac_v, out_v, offs_v):
        @plsc.parallel_loop(0, 8, unroll=4)
        def mbody(m):
            sbase = m * 128
            for gg in range(8):
                old = offs_v[pl.ds(sbase + gg * 16, 16)]
                sgn = sgn_v[pl.ds(sbase + gg * 16, 16)]
                frac = frac_v[pl.ds(sbase + gg * 16, 16)]
                idx = (frac * 15.0).astype(jnp.int32)
                offs = (lane_off + (sbase + gg * 2048)) + idx
                plsc.store_scatter(out_v, [old], zvec, mask=old != offs)
                offs_v[pl.ds(sbase + gg * 16, 16)] = offs
                plsc.store_scatter(out_v, [offs], sgn)

    def out_slice(k):
        return out_hbm.at[pl.ds(out_w0 + CHUNK_OUT * k, CHUNK_OUT)]

    def stage_slice(s):
        return stage.at[pl.ds((s * 16 + b) * CHUNK_OUT, CHUNK_OUT)]

    def a_chunk(k, first=False, prefetch=True):
        if not first:
            pltpu.make_async_copy(out_a, out_slice(k - 2), sem_a).wait()
        in_wait(k, sgn_a, frac_a, isem_a)
        merged_pass(sgn_a, frac_a, out_a, offs_a)
        if prefetch:
            in_issue(k + 2, sgn_a, frac_a, isem_a)
        pltpu.async_copy(out_a, out_slice(k), sem_a)

    def b_chunk(k, s, first=False, second=False, prefetch=True):
        os = 1 - s
        if not first:
            pltpu.make_async_copy(out_b, stage_slice(os), semx[os]).wait()
            pltpu.async_copy(stage_slice(os), out_slice(k - 2), semy[os])
        in_wait(k, sgn_b, frac_b, isem_b)
        merged_pass(sgn_b, frac_b, out_b, offs_b)
        if prefetch:
            in_issue(k + 2, sgn_b, frac_b, isem_b)
        if not (first or second):
            pltpu.make_async_copy(stage_slice(s), out_slice(k - 4), semy[s]).wait()
        pltpu.async_copy(out_b, stage_slice(s), semx[s])

    in_issue(0, sgn_a, frac_a, isem_a)
    in_issue(1, sgn_b, frac_b, isem_b)
    init_offs(offs_a)
    init_offs(offs_b)
    zero_buf(out_a)
    a_chunk(0, first=True)
    zero_buf(out_b)
    b_chunk(1, 0, first=True)
    a_chunk(2)
    b_chunk(3, 1, second=True)

    def quad(u, _):
        k = 4 * u
        a_chunk(k)
        b_chunk(k + 1, 0)
        a_chunk(k + 2)
        b_chunk(k + 3, 1)
        return 0

    lax.fori_loop(1, NCHUNK // 4 - 1, quad, 0)

    a_chunk(NCHUNK - 4)
    b_chunk(NCHUNK - 3, 0)
    a_chunk(NCHUNK - 2, prefetch=False)
    b_chunk(NCHUNK - 1, 1, prefetch=False)

    pltpu.make_async_copy(out_b, stage_slice(1), semx1).wait()
    pltpu.async_copy(stage_slice(1), out_slice(NCHUNK - 1), semy1)
    pltpu.make_async_copy(out_a, out_slice(NCHUNK - 2), sem_a).wait()
    pltpu.make_async_copy(stage_slice(0), out_slice(NCHUNK - 3), semy0).wait()
    pltpu.make_async_copy(stage_slice(1), out_slice(NCHUNK - 1), semy1).wait()


def kernel(inputs):
    flat_in = (inputs.transpose(0, 2, 1)
               .reshape(B, 4, 8, 32, 128)
               .transpose(0, 1, 3, 2, 4)
               .reshape(IN_WORDS))
    flat_out = _velvet_sc(flat_in)
    return (flat_out.reshape(B, 2, 512, 8, 128)
            .transpose(0, 2, 4, 1, 3)
            .reshape(B, N, C))
